# baseline (speedup 1.0000x reference)
import jax
import jax.numpy as jnp
from jax.experimental import pallas as pl
from jax.experimental.pallas import tpu as pltpu

A = 4
K = 1024
D = 64
B = 16
T = 576
N = B * T


def _vq_body(x_ref, w_ref, q_ref, idx_ref, loss_ref, perp_ref, counts_ref,
             sw_ref):
    b = pl.program_id(0)

    @pl.when(b == 0)
    def _reset():
        counts_ref[...] = jnp.zeros_like(counts_ref)
        for a in range(A):
            loss_ref[a] = 0.0
            w = w_ref[a]
            sw_ref[a:a + 1, :] = jnp.sum(w * w, axis=1)[None, :]

    for a in range(A):
        x = x_ref[0, :, T * a:T * (a + 1)]
        w = w_ref[a]
        xt = x.T
        mm2 = jax.lax.dot_general(xt + xt, w, (((1,), (1,)), ((), ())),
                                  preferred_element_type=jnp.float32)
        sx = jnp.sum(xt * xt, axis=1, keepdims=True)
        sw = sw_ref[a:a + 1, :]
        dist = (sx + sw) - mm2

        m = jnp.min(dist, axis=1, keepdims=True)
        idx = jnp.argmin(dist, axis=1)[:, None]
        lane = jax.lax.broadcasted_iota(jnp.int32, (T, K), 1)

        oh = (lane == idx).astype(jnp.float32)
        q = jax.lax.dot_general(w, oh, (((0,), (1,)), ((), ())),
                                preferred_element_type=jnp.float32)
        q_ref[0, :, T * a:T * (a + 1)] = q
        idx_ref[0, :, a:a + 1] = idx

        counts_ref[a:a + 1, :] += jnp.sum(oh, axis=0, keepdims=True)
        loss_ref[a] += jnp.sum(m)

    @pl.when(b == B - 1)
    def _finalize():
        p = counts_ref[...] / N
        ent = jnp.sum(p * jnp.log(p + 1e-10), axis=1)
        for a in range(A):
            perp_ref[a] = jnp.exp(-ent[a])


def _vq(x2, emb):
    return pl.pallas_call(
        _vq_body,
        grid=(B,),
        in_specs=[
            pl.BlockSpec((1, D, A * T), lambda b: (b, 0, 0)),
            pl.BlockSpec((A, K, D), lambda b: (0, 0, 0)),
        ],
        out_specs=[
            pl.BlockSpec((1, D, A * T), lambda b: (b, 0, 0)),
            pl.BlockSpec((1, T, A), lambda b: (b, 0, 0)),
            pl.BlockSpec(memory_space=pltpu.SMEM),
            pl.BlockSpec(memory_space=pltpu.SMEM),
        ],
        out_shape=[
            jax.ShapeDtypeStruct((B, D, A * T), jnp.float32),
            jax.ShapeDtypeStruct((B, T, A), jnp.int32),
            jax.ShapeDtypeStruct((A,), jnp.float32),
            jax.ShapeDtypeStruct((A,), jnp.float32),
        ],
        scratch_shapes=[
            pltpu.VMEM((A, K), jnp.float32),
            pltpu.VMEM((A, K), jnp.float32),
        ],
    )(x2, emb)


def kernel(inputs, emb):
    x2 = inputs.reshape(B, D, A * T)
    q2, idx2, loss_sums, perps = _vq(x2, emb)
    quantized = q2.reshape(B, D, A, T)
    encoding_indices = idx2.reshape(N, A, 1)
    l = loss_sums / jnp.float32(N * D)
    q_loss = jnp.sum(l) / A
    e_loss = jnp.sum(0.25 * l) / A
    perplexity = jnp.sum(perps) / A
    return q_loss, e_loss, quantized, perplexity, encoding_indices


# speedup vs baseline: 1.4638x; 1.4638x over previous
import jax
import jax.numpy as jnp
from jax.experimental import pallas as pl
from jax.experimental.pallas import tpu as pltpu

A = 4
K = 1024
D = 64
B = 16
T = 576
N = B * T


def _vq_body(x_ref, w_ref, q_ref, idx_ref, loss_ref, perp_ref, counts_ref,
             sw_ref):
    b = pl.program_id(0)

    @pl.when(b == 0)
    def _reset():
        counts_ref[...] = jnp.zeros_like(counts_ref)
        for a in range(A):
            loss_ref[a] = 0.0
            w = w_ref[a]
            sw_ref[a:a + 1, :] = jnp.sum(w * w, axis=1)[None, :]

    for a in range(A):
        x = x_ref[0, :, T * a:T * (a + 1)]
        w = w_ref[a]
        xt = x.T
        mm2 = jax.lax.dot_general(xt + xt, w, (((1,), (1,)), ((), ())),
                                  preferred_element_type=jnp.float32)
        sx = jnp.sum(xt * xt, axis=1, keepdims=True)
        sw = sw_ref[a:a + 1, :]
        dist = (sx + sw) - mm2

        m = jnp.min(dist, axis=1, keepdims=True)
        lane = jax.lax.broadcasted_iota(jnp.int32, (T, K), 1)
        idx = jnp.min(jnp.where(dist == m, lane, K), axis=1, keepdims=True)

        oh = (lane == idx).astype(jnp.float32)
        q = jax.lax.dot_general(w, oh, (((0,), (1,)), ((), ())),
                                preferred_element_type=jnp.float32)
        q_ref[0, :, T * a:T * (a + 1)] = q
        idx_ref[0, :, a:a + 1] = idx

        counts_ref[a:a + 1, :] += jnp.sum(oh, axis=0, keepdims=True)
        loss_ref[a] += jnp.sum(m)

    @pl.when(b == B - 1)
    def _finalize():
        p = counts_ref[...] / N
        ent = jnp.sum(p * jnp.log(p + 1e-10), axis=1)
        for a in range(A):
            perp_ref[a] = jnp.exp(-ent[a])


def _vq(x2, emb):
    return pl.pallas_call(
        _vq_body,
        grid=(B,),
        in_specs=[
            pl.BlockSpec((1, D, A * T), lambda b: (b, 0, 0)),
            pl.BlockSpec((A, K, D), lambda b: (0, 0, 0)),
        ],
        out_specs=[
            pl.BlockSpec((1, D, A * T), lambda b: (b, 0, 0)),
            pl.BlockSpec((1, T, A), lambda b: (b, 0, 0)),
            pl.BlockSpec(memory_space=pltpu.SMEM),
            pl.BlockSpec(memory_space=pltpu.SMEM),
        ],
        out_shape=[
            jax.ShapeDtypeStruct((B, D, A * T), jnp.float32),
            jax.ShapeDtypeStruct((B, T, A), jnp.int32),
            jax.ShapeDtypeStruct((A,), jnp.float32),
            jax.ShapeDtypeStruct((A,), jnp.float32),
        ],
        scratch_shapes=[
            pltpu.VMEM((A, K), jnp.float32),
            pltpu.VMEM((A, K), jnp.float32),
        ],
    )(x2, emb)


def kernel(inputs, emb):
    x2 = inputs.reshape(B, D, A * T)
    q2, idx2, loss_sums, perps = _vq(x2, emb)
    quantized = q2.reshape(B, D, A, T)
    encoding_indices = idx2.reshape(N, A, 1)
    l = loss_sums / jnp.float32(N * D)
    q_loss = jnp.sum(l) / A
    e_loss = jnp.sum(0.25 * l) / A
    perplexity = jnp.sum(perps) / A
    return q_loss, e_loss, quantized, perplexity, encoding_indices


# speedup vs baseline: 1.5154x; 1.0352x over previous
import jax
import jax.numpy as jnp
from jax.experimental import pallas as pl
from jax.experimental.pallas import tpu as pltpu

A = 4
K = 1024
D = 64
B = 16
T = 576
N = B * T


def _vq_body(x_ref, w_ref, q_ref, idx_ref, loss_ref, perp_ref, counts_ref,
             sw_ref):
    b = pl.program_id(0)

    @pl.when(b == 0)
    def _reset():
        counts_ref[...] = jnp.zeros_like(counts_ref)
        for a in range(A):
            loss_ref[a] = 0.0
            w = w_ref[a]
            sw_ref[a:a + 1, :] = jnp.sum(w * w, axis=1)[None, :]

    for a in range(A):
        x = x_ref[0, :, T * a:T * (a + 1)]
        w = w_ref[a]
        xt = x.T
        mm2 = jax.lax.dot_general(xt + xt, w, (((1,), (1,)), ((), ())),
                                  preferred_element_type=jnp.float32)
        sx = jnp.sum(xt * xt, axis=1, keepdims=True)
        sw = sw_ref[a:a + 1, :]
        dist = (sx + sw) - mm2

        m = jnp.min(dist, axis=1, keepdims=True)
        lane_f = jax.lax.broadcasted_iota(jnp.int32, (T, K), 1).astype(
            jnp.float32)
        idx_f = jnp.min(jnp.where(dist == m, lane_f, jnp.float32(K)),
                        axis=1, keepdims=True)

        oh = (lane_f == idx_f).astype(jnp.float32)
        q = jax.lax.dot_general(w, oh, (((0,), (1,)), ((), ())),
                                preferred_element_type=jnp.float32)
        q_ref[0, :, T * a:T * (a + 1)] = q
        idx_ref[0, :, a:a + 1] = idx_f.astype(jnp.int32)

        ones_row = jnp.ones((1, T), dtype=jnp.float32)
        cnt = jax.lax.dot_general(ones_row, oh, (((1,), (0,)), ((), ())),
                                  preferred_element_type=jnp.float32)
        counts_ref[a:a + 1, :] += cnt
        loss_ref[a] += jnp.sum(m)

    @pl.when(b == B - 1)
    def _finalize():
        p = counts_ref[...] / N
        ent = jnp.sum(p * jnp.log(p + 1e-10), axis=1)
        for a in range(A):
            perp_ref[a] = jnp.exp(-ent[a])


def _vq(x2, emb):
    return pl.pallas_call(
        _vq_body,
        grid=(B,),
        in_specs=[
            pl.BlockSpec((1, D, A * T), lambda b: (b, 0, 0)),
            pl.BlockSpec((A, K, D), lambda b: (0, 0, 0)),
        ],
        out_specs=[
            pl.BlockSpec((1, D, A * T), lambda b: (b, 0, 0)),
            pl.BlockSpec((1, T, A), lambda b: (b, 0, 0)),
            pl.BlockSpec(memory_space=pltpu.SMEM),
            pl.BlockSpec(memory_space=pltpu.SMEM),
        ],
        out_shape=[
            jax.ShapeDtypeStruct((B, D, A * T), jnp.float32),
            jax.ShapeDtypeStruct((B, T, A), jnp.int32),
            jax.ShapeDtypeStruct((A,), jnp.float32),
            jax.ShapeDtypeStruct((A,), jnp.float32),
        ],
        scratch_shapes=[
            pltpu.VMEM((A, K), jnp.float32),
            pltpu.VMEM((A, K), jnp.float32),
        ],
    )(x2, emb)


def kernel(inputs, emb):
    x2 = inputs.reshape(B, D, A * T)
    q2, idx2, loss_sums, perps = _vq(x2, emb)
    quantized = q2.reshape(B, D, A, T)
    encoding_indices = idx2.reshape(N, A, 1)
    l = loss_sums / jnp.float32(N * D)
    q_loss = jnp.sum(l) / A
    e_loss = jnp.sum(0.25 * l) / A
    perplexity = jnp.sum(perps) / A
    return q_loss, e_loss, quantized, perplexity, encoding_indices
